# baseline (speedup 1.0000x reference)
import functools

import jax
import jax.numpy as jnp
from jax import lax
from jax.experimental import pallas as pl
from jax.experimental.pallas import tpu as pltpu
from jax.experimental.pallas import tpu_sc as plsc

N = 10000
E = 320000
D = 128
H = 64
HH = H // 2

NC = 2
NS = 16
NW = NC * NS
EPT = E // NW
CP = 80
NCH2 = 2 * EPT // CP
NBUF = 5
NV = EPT // 16
REG = 640

_MESH = plsc.VectorSubcoreMesh(core_axis_name="c", subcore_axis_name="s")
_NOTC = pltpu.CompilerParams(use_tc_tiling_on_sc=False,
                             needs_layout_passes=False)

_Z16 = functools.partial(jnp.zeros, (16,), jnp.float32)


def _zero2(buf, rows, cols):
    def zb(r, c):
        for k in range(cols // 16):
            buf[r, pl.ds(k * 16, 16)] = _Z16()
        return c
    lax.fori_loop(0, rows, zb, 0)


def _zero1_loop(buf, nvec):
    def zb(i, c):
        buf[pl.ds(i * 16, 16)] = _Z16()
        return c
    lax.fori_loop(0, nvec, zb, 0)


def _hist_reduce_out(stg, hbuf, obuf, out_hbm, cid, sid):
    def _region(r0, nv16):
        pltpu.sync_copy(stg.at[:, pl.ds(r0, nv16 * 16)],
                        hbuf.at[:, pl.ds(0, nv16 * 16)])

        def red(i, c):
            s = hbuf[0, pl.ds(i * 16, 16)]
            for k in range(1, NS):
                s = s + hbuf[k, pl.ds(i * 16, 16)]
            obuf[pl.ds(i * 16, 16)] = s
            return c
        lax.fori_loop(0, nv16, red, 0)
        pltpu.sync_copy(obuf.at[pl.ds(0, nv16 * 16)],
                        out_hbm.at[pl.ds(cid * N + r0, nv16 * 16)])

    @pl.when(sid < NS - 1)
    def _():
        _region(sid * REG, REG // 16)

    @pl.when(sid == NS - 1)
    def _():
        _region((NS - 1) * REG, (N - (NS - 1) * REG) // 16)


@functools.partial(
    pl.kernel,
    mesh=_MESH,
    compiler_params=_NOTC,
    out_type=jax.ShapeDtypeStruct((NC * N,), jnp.float32),
    scratch_types=[
        pltpu.VMEM((EPT,), jnp.int32),
        pltpu.VMEM((N,), jnp.float32),
        pltpu.VMEM((NS, REG), jnp.float32),
        pltpu.VMEM((REG,), jnp.float32),
        pltpu.VMEM_SHARED((NS, N), jnp.float32),
    ],
)
def _deg_sc(e5_hbm, out_hbm, didx, hist, hbuf, obuf, stg):
    cid = lax.axis_index("c")
    sid = lax.axis_index("s")
    wid = cid * NS + sid

    pltpu.sync_copy(e5_hbm.at[1, wid], didx)
    _zero1_loop(hist, N // 16)

    ones = jnp.full((16,), 1.0, jnp.float32)

    def body(g, c):
        for u in range(5):
            di = didx[pl.ds((g * 5 + u) * 16, 16)]
            plsc.addupdate_scatter(hist, [di], ones)
        return c

    lax.fori_loop(0, NV // 5, body, 0)

    pltpu.sync_copy(hist, stg.at[sid])
    plsc.subcore_barrier()
    _hist_reduce_out(stg, hbuf, obuf, out_hbm, cid, sid)


@functools.partial(
    pl.kernel,
    mesh=_MESH,
    compiler_params=_NOTC,
    out_type=jax.ShapeDtypeStruct((NC * N,), jnp.float32),
    scratch_types=(
        [pltpu.VMEM((2 * EPT,), jnp.int32),
         pltpu.VMEM((2 * EPT,), jnp.int32)]
        + [pltpu.VMEM((CP, HH), jnp.float32) for _ in range(NBUF)]
        + [pltpu.VMEM((REG, HH), jnp.float32),
           pltpu.VMEM((REG, HH), jnp.float32),
           pltpu.VMEM((REG, 1), jnp.float32),
           pltpu.VMEM((REG,), jnp.float32),
           pltpu.VMEM((HH,), jnp.float32),
           pltpu.VMEM((HH,), jnp.float32),
           pltpu.VMEM_SHARED((N, HH), jnp.float32)]
        + [pltpu.SemaphoreType.DMA for _ in range(2 * NBUF)]
    ),
)
def _prop64_sc(y1t_hbm, e5_hbm, dinv_hbm, b1t_hbm, w2t_hbm, out_hbm,
               sidx, didx, *rest):
    rows = rest[:NBUF]
    ca, cy, dv, ob, bb, wv, acc = rest[NBUF:NBUF + 7]
    sem_g = rest[NBUF + 7:2 * NBUF + 7]
    sem_s = rest[2 * NBUF + 7:]
    cid = lax.axis_index("c")
    sid = lax.axis_index("s")

    pltpu.sync_copy(e5_hbm.at[0, 2 * sid], sidx.at[pl.ds(0, EPT)])
    pltpu.sync_copy(e5_hbm.at[0, 2 * sid + 1], sidx.at[pl.ds(EPT, EPT)])
    pltpu.sync_copy(e5_hbm.at[1, 2 * sid], didx.at[pl.ds(0, EPT)])
    pltpu.sync_copy(e5_hbm.at[1, 2 * sid + 1], didx.at[pl.ds(EPT, EPT)])
    pltpu.sync_copy(b1t_hbm.at[cid], bb)
    pltpu.sync_copy(w2t_hbm.at[cid], wv)

    _zero2(rows[0], CP, HH)
    r0i = sid * REG

    @pl.when(sid < NS - 1)
    def _():
        for k in range(REG // CP):
            pltpu.sync_copy(rows[0], acc.at[pl.ds(r0i + k * CP, CP)])

    @pl.when(sid == NS - 1)
    def _():
        for k in range((N - (NS - 1) * REG) // CP):
            pltpu.sync_copy(rows[0], acc.at[pl.ds(r0i + k * CP, CP)])

    plsc.subcore_barrier()

    ytab = y1t_hbm.at[cid]

    def body(g, carry):
        gd = []
        for b in range(NBUF):
            @pl.when(g > 0)
            def _(b=b):
                pltpu.make_async_copy(rows[b],
                                      acc.at[didx.at[pl.ds(0, CP)]],
                                      sem_s[b]).wait()
            gd.append(pltpu.async_copy(
                ytab.at[sidx.at[pl.ds((g * NBUF + b) * CP, CP)]],
                rows[b], sem_g[b]))
        for b in range(NBUF):
            gd[b].wait()
            pltpu.async_copy(rows[b],
                             acc.at[didx.at[pl.ds((g * NBUF + b) * CP, CP)]],
                             sem_s[b], add=True)
        return carry

    lax.fori_loop(0, NCH2 // NBUF, body, 0)
    for b in range(NBUF):
        pltpu.make_async_copy(rows[b], acc.at[didx.at[pl.ds(0, CP)]],
                              sem_s[b]).wait()

    plsc.subcore_barrier()

    iota = lax.iota(jnp.int32, 16)
    wv0 = wv[pl.ds(0, 16)]
    wv1 = wv[pl.ds(16, 16)]
    bb0 = bb[pl.ds(0, 16)]
    bb1 = bb[pl.ds(16, 16)]

    def _epi(r0, nv16):
        pltpu.sync_copy(acc.at[pl.ds(r0, nv16 * 16)],
                        ca.at[pl.ds(0, nv16 * 16)])
        pltpu.sync_copy(ytab.at[pl.ds(r0, nv16 * 16)],
                        cy.at[pl.ds(0, nv16 * 16)])
        pltpu.sync_copy(dinv_hbm.at[pl.ds(r0, nv16 * 16)],
                        dv.at[pl.ds(0, nv16 * 16)])

        def grp(g, c):
            res = _Z16()
            for j in range(16):
                r = g * 16 + j
                splat = plsc.load_gather(
                    dv, [jnp.full((16,), r, jnp.int32),
                         jnp.zeros((16,), jnp.int32)])
                a0 = ca[r, pl.ds(0, 16)] + cy[r, pl.ds(0, 16)]
                a1 = ca[r, pl.ds(16, 16)] + cy[r, pl.ds(16, 16)]
                h0 = jnp.maximum(a0 * splat + bb0, 0.0)
                h1 = jnp.maximum(a1 * splat + bb1, 0.0)
                t = (h0 * wv0 + h1 * wv1) * splat
                s2 = jnp.sum(t, axis=0)
                res = jnp.where(iota == j, s2, res)
            ob[pl.ds(g * 16, 16)] = res
            return c

        lax.fori_loop(0, nv16, grp, 0)
        pltpu.sync_copy(ob.at[pl.ds(0, nv16 * 16)],
                        out_hbm.at[pl.ds(cid * N + r0, nv16 * 16)])

    @pl.when(sid < NS - 1)
    def _():
        _epi(sid * REG, REG // 16)

    @pl.when(sid == NS - 1)
    def _():
        _epi((NS - 1) * REG, (N - (NS - 1) * REG) // 16)


@functools.partial(
    pl.kernel,
    mesh=_MESH,
    compiler_params=_NOTC,
    out_type=jax.ShapeDtypeStruct((NC * N,), jnp.float32),
    scratch_types=[
        pltpu.VMEM((EPT,), jnp.int32),
        pltpu.VMEM((EPT,), jnp.int32),
        pltpu.VMEM((N,), jnp.float32),
        pltpu.VMEM((N,), jnp.float32),
        pltpu.VMEM((N,), jnp.float32),
        pltpu.VMEM((NS, REG), jnp.float32),
        pltpu.VMEM((REG,), jnp.float32),
        pltpu.VMEM_SHARED((NS, N), jnp.float32),
    ],
)
def _prop1_sc(y2p_hbm, e5_hbm, out_hbm, sidx, didx, yva, yvb,
              hist, hbuf, obuf, stg):
    cid = lax.axis_index("c")
    sid = lax.axis_index("s")
    wid = cid * NS + sid

    pltpu.sync_copy(e5_hbm.at[0, wid], sidx)
    pltpu.sync_copy(e5_hbm.at[1, wid], didx)
    pltpu.sync_copy(y2p_hbm.at[pl.ds(0, N)], yva)
    pltpu.sync_copy(y2p_hbm.at[pl.ds(N, N)], yvb)
    _zero1_loop(hist, N // 16)

    def body(g, c):
        for u in range(5):
            j16 = (g * 5 + u) * 16
            si = sidx[pl.ds(j16, 16)]
            di = didx[pl.ds(j16, 16)]
            vals = plsc.load_gather(yva, [si]) + plsc.load_gather(yvb, [si])
            plsc.addupdate_scatter(hist, [di], vals)
        return c

    lax.fori_loop(0, NV // 5, body, 0)

    pltpu.sync_copy(hist, stg.at[sid])
    plsc.subcore_barrier()
    _hist_reduce_out(stg, hbuf, obuf, out_hbm, cid, sid)


def _prep_body(x_ref, w_ref, degp_ref, dinv_ref, y1t_ref):
    deg = degp_ref[pl.ds(0, N)] + degp_ref[pl.ds(N, N)] + 1.0
    dinv = lax.rsqrt(deg)
    dinv_ref[...] = dinv[:, None]
    xw = jnp.dot(x_ref[...], w_ref[...], preferred_element_type=jnp.float32)
    y1 = xw * dinv[:, None]
    y1t_ref[0] = y1[:, :HH]
    y1t_ref[1] = y1[:, HH:]


def _prep(x, W1, degp):
    return pl.pallas_call(
        _prep_body,
        out_shape=[jax.ShapeDtypeStruct((N, 1), jnp.float32),
                   jax.ShapeDtypeStruct((NC, N, HH), jnp.float32)],
    )(x, W1, degp)


def _fin_body(acc2_ref, y2p_ref, dinv_ref, b2_ref, out_ref):
    a = acc2_ref[pl.ds(0, N)] + acc2_ref[pl.ds(N, N)]
    y2 = y2p_ref[pl.ds(0, N)] + y2p_ref[pl.ds(N, N)]
    out_ref[...] = ((a + y2) * dinv_ref[:, 0] + b2_ref[0])[:, None]


def _fin(acc2, y2p, dinv, b2):
    return pl.pallas_call(
        _fin_body,
        out_shape=jax.ShapeDtypeStruct((N, 1), jnp.float32),
    )(acc2, y2p, dinv, b2)


def kernel(x, e, W1, b1, W2, b2):
    e5 = jnp.reshape(e, (2, NW, EPT))
    b1t = jnp.reshape(b1, (NC, HH))
    w2t = jnp.reshape(W2, (NC, HH))

    degp = _deg_sc(e5)
    dinv, y1t = _prep(x, W1, degp)
    y2p = _prop64_sc(y1t, e5, dinv, b1t, w2t)
    acc2 = _prop1_sc(y2p, e5)
    out = _fin(acc2, y2p, dinv, b2)
    return out


# speedup vs baseline: 1.1884x; 1.1884x over previous
import functools

import jax
import jax.numpy as jnp
from jax import lax
from jax.experimental import pallas as pl
from jax.experimental.pallas import tpu as pltpu
from jax.experimental.pallas import tpu_sc as plsc

N = 10000
E = 320000
D = 128
H = 64

NC = 2
NS = 16
NW = NC * NS
EPT = E // NW
CP = 80
NCH = EPT // CP
NBUF = 5
WCH = 125
NV = EPT // 16
REG = 640

_MESH = plsc.VectorSubcoreMesh(core_axis_name="c", subcore_axis_name="s")
_NOTC = pltpu.CompilerParams(use_tc_tiling_on_sc=False,
                             needs_layout_passes=False)

_Z16 = functools.partial(jnp.zeros, (16,), jnp.float32)


def _zero2(buf, rows):
    def zb(r, c):
        for k in range(H // 16):
            buf[r, pl.ds(k * 16, 16)] = _Z16()
        return c
    lax.fori_loop(0, rows, zb, 0)


def _zero1_loop(buf, nvec):
    def zb(i, c):
        buf[pl.ds(i * 16, 16)] = _Z16()
        return c
    lax.fori_loop(0, nvec, zb, 0)


def _hist_reduce_out(stg, hbuf, obuf, out_hbm, cid, sid):
    def _region(r0, nv16):
        pltpu.sync_copy(stg.at[:, pl.ds(r0, nv16 * 16)],
                        hbuf.at[:, pl.ds(0, nv16 * 16)])

        def red(i, c):
            s = hbuf[0, pl.ds(i * 16, 16)]
            for k in range(1, NS):
                s = s + hbuf[k, pl.ds(i * 16, 16)]
            obuf[pl.ds(i * 16, 16)] = s
            return c
        lax.fori_loop(0, nv16, red, 0)
        pltpu.sync_copy(obuf.at[pl.ds(0, nv16 * 16)],
                        out_hbm.at[pl.ds(cid * N + r0, nv16 * 16)])

    @pl.when(sid < NS - 1)
    def _():
        _region(sid * REG, REG // 16)

    @pl.when(sid == NS - 1)
    def _():
        _region((NS - 1) * REG, (N - (NS - 1) * REG) // 16)


@functools.partial(
    pl.kernel,
    mesh=_MESH,
    compiler_params=_NOTC,
    out_type=jax.ShapeDtypeStruct((NC * N,), jnp.float32),
    scratch_types=[
        pltpu.VMEM((EPT,), jnp.int32),
        pltpu.VMEM((N,), jnp.float32),
        pltpu.VMEM((NS, REG), jnp.float32),
        pltpu.VMEM((REG,), jnp.float32),
        pltpu.VMEM_SHARED((NS, N), jnp.float32),
        pltpu.SemaphoreType.DMA,
    ],
)
def _deg_sc(e5_hbm, out_hbm, didx, hist, hbuf, obuf, stg, sem):
    cid = lax.axis_index("c")
    sid = lax.axis_index("s")
    wid = cid * NS + sid

    d0 = pltpu.async_copy(e5_hbm.at[1, wid], didx, sem)
    _zero1_loop(hist, N // 16)
    d0.wait()

    ones = jnp.full((16,), 1.0, jnp.float32)

    def body(g, c):
        for u in range(5):
            di = didx[pl.ds((g * 5 + u) * 16, 16)]
            plsc.addupdate_scatter(hist, [di], ones)
        return c

    lax.fori_loop(0, NV // 5, body, 0)

    pltpu.sync_copy(hist, stg.at[sid])
    plsc.subcore_barrier()
    _hist_reduce_out(stg, hbuf, obuf, out_hbm, cid, sid)


@functools.partial(
    pl.kernel,
    mesh=_MESH,
    compiler_params=_NOTC,
    out_type=jax.ShapeDtypeStruct((NC, N, H), jnp.float32),
    scratch_types=(
        [pltpu.VMEM((EPT,), jnp.int32),
         pltpu.VMEM((EPT,), jnp.int32)]
        + [pltpu.VMEM((WCH, H), jnp.float32) for _ in range(NBUF)]
        + [pltpu.VMEM_SHARED((N, H), jnp.float32)]
        + [pltpu.SemaphoreType.DMA for _ in range(2 * NBUF)]
    ),
)
def _prop64_sc(y_hbm, e5_hbm, out_hbm, sidx, didx, *rest):
    rows = rest[:NBUF]
    acc = rest[NBUF]
    sem_g = rest[NBUF + 1:2 * NBUF + 1]
    sem_s = rest[2 * NBUF + 1:]
    cid = lax.axis_index("c")
    sid = lax.axis_index("s")
    wid = cid * NS + sid
    nrt = N // NS

    d0 = pltpu.async_copy(e5_hbm.at[0, wid], sidx, sem_g[0])
    d1 = pltpu.async_copy(e5_hbm.at[1, wid], didx, sem_g[1])
    _zero2(rows[0], WCH)
    d0.wait()
    d1.wait()
    for k in range(nrt // WCH):
        pltpu.sync_copy(rows[0], acc.at[pl.ds(sid * nrt + k * WCH, WCH)])

    plsc.subcore_barrier()

    def body(g, carry):
        gd = []
        for b in range(NBUF):
            @pl.when(g > 0)
            def _(b=b):
                pltpu.make_async_copy(rows[b].at[pl.ds(0, CP)],
                                      acc.at[didx.at[pl.ds(0, CP)]],
                                      sem_s[b]).wait()
            gd.append(pltpu.async_copy(y_hbm.at[sidx.at[pl.ds((g * NBUF + b) * CP, CP)]],
                                       rows[b].at[pl.ds(0, CP)], sem_g[b]))
        for b in range(NBUF):
            gd[b].wait()
            pltpu.async_copy(rows[b].at[pl.ds(0, CP)],
                             acc.at[didx.at[pl.ds((g * NBUF + b) * CP, CP)]],
                             sem_s[b], add=True)
        return carry

    lax.fori_loop(0, NCH // NBUF, body, 0)
    for b in range(NBUF):
        pltpu.make_async_copy(rows[b].at[pl.ds(0, CP)],
                              acc.at[didx.at[pl.ds(0, CP)]], sem_s[b]).wait()

    plsc.subcore_barrier()

    wo = []
    for k in range(nrt // WCH):
        r0 = sid * nrt + k * WCH
        wo.append(pltpu.async_copy(acc.at[pl.ds(r0, WCH)], rows[k], sem_g[k]))
    wo2 = []
    for k in range(nrt // WCH):
        r0 = sid * nrt + k * WCH
        wo[k].wait()
        wo2.append(pltpu.async_copy(rows[k], out_hbm.at[cid, pl.ds(r0, WCH)],
                                    sem_s[k]))
    for d in wo2:
        d.wait()


@functools.partial(
    pl.kernel,
    mesh=_MESH,
    compiler_params=_NOTC,
    out_type=jax.ShapeDtypeStruct((NC * N,), jnp.float32),
    scratch_types=[
        pltpu.VMEM((EPT,), jnp.int32),
        pltpu.VMEM((EPT,), jnp.int32),
        pltpu.VMEM((N,), jnp.float32),
        pltpu.VMEM((N,), jnp.float32),
        pltpu.VMEM((NS, REG), jnp.float32),
        pltpu.VMEM((REG,), jnp.float32),
        pltpu.VMEM_SHARED((NS, N), jnp.float32),
        pltpu.SemaphoreType.DMA,
        pltpu.SemaphoreType.DMA,
        pltpu.SemaphoreType.DMA,
    ],
)
def _prop1_sc(y_hbm, e5_hbm, out_hbm, sidx, didx, yv, hist, hbuf, obuf, stg,
              s0, s1, s2):
    cid = lax.axis_index("c")
    sid = lax.axis_index("s")
    wid = cid * NS + sid

    d0 = pltpu.async_copy(e5_hbm.at[0, wid], sidx, s0)
    d1 = pltpu.async_copy(e5_hbm.at[1, wid], didx, s1)
    d2 = pltpu.async_copy(y_hbm, yv, s2)
    _zero1_loop(hist, N // 16)
    d0.wait()
    d1.wait()
    d2.wait()

    def body(g, c):
        for u in range(5):
            j16 = (g * 5 + u) * 16
            si = sidx[pl.ds(j16, 16)]
            di = didx[pl.ds(j16, 16)]
            vals = plsc.load_gather(yv, [si])
            plsc.addupdate_scatter(hist, [di], vals)
        return c

    lax.fori_loop(0, NV // 5, body, 0)

    pltpu.sync_copy(hist, stg.at[sid])
    plsc.subcore_barrier()
    _hist_reduce_out(stg, hbuf, obuf, out_hbm, cid, sid)


def _prep_body(x_ref, w_ref, degp_ref, dinv_ref, y1_ref):
    deg = degp_ref[pl.ds(0, N)] + degp_ref[pl.ds(N, N)] + 1.0
    dinv = lax.rsqrt(deg)
    dinv_ref[...] = dinv
    xw = jnp.dot(x_ref[...], w_ref[...], preferred_element_type=jnp.float32)
    y1_ref[...] = xw * dinv[:, None]


def _prep(x, W1, degp):
    return pl.pallas_call(
        _prep_body,
        out_shape=[jax.ShapeDtypeStruct((N,), jnp.float32),
                   jax.ShapeDtypeStruct((N, H), jnp.float32)],
    )(x, W1, degp)


def _l2_body(accp_ref, y1_ref, dinv_ref, w2_ref, b1_ref, y2_ref):
    a = accp_ref[0] + accp_ref[1] + y1_ref[...]
    h = jnp.maximum(a * dinv_ref[...][:, None] + b1_ref[...], 0.0)
    hw = jnp.dot(h, w2_ref[...], preferred_element_type=jnp.float32)
    y2_ref[...] = hw[:, 0] * dinv_ref[...]


def _l2(accp, y1, dinv, W2, b1_row):
    return pl.pallas_call(
        _l2_body,
        out_shape=jax.ShapeDtypeStruct((N,), jnp.float32),
    )(accp, y1, dinv, W2, b1_row)


def _fin_body(acc2_ref, y2_ref, dinv_ref, b2_ref, out_ref):
    a = acc2_ref[pl.ds(0, N)] + acc2_ref[pl.ds(N, N)]
    out_ref[...] = ((a + y2_ref[...]) * dinv_ref[...] + b2_ref[0])[:, None]


def _fin(acc2, y2, dinv, b2):
    return pl.pallas_call(
        _fin_body,
        out_shape=jax.ShapeDtypeStruct((N, 1), jnp.float32),
    )(acc2, y2, dinv, b2)


def kernel(x, e, W1, b1, W2, b2):
    e5 = jnp.reshape(e, (2, NW, EPT))

    degp = _deg_sc(e5)
    dinv, y1 = _prep(x, W1, degp)
    accp = _prop64_sc(y1, e5)
    y2 = _l2(accp, y1, dinv, W2, jnp.reshape(b1, (1, H)))
    acc2 = _prop1_sc(y2, e5)
    out = _fin(acc2, y2, dinv, b2)
    return out
